# baseline (speedup 1.0000x reference)
import functools

import jax
import jax.numpy as jnp
from jax import lax
from jax.experimental import pallas as pl
from jax.experimental.pallas import tpu as pltpu
from jax.experimental.pallas import tpu_sc as plsc

_EPS = 1e-5

_B = 16384
_E = 64
_F = 128
_H = 256
_O = 128


_NC = 2
_NS = 16
_NW = _NC * _NS
_CHUNK = 128
_CPW = _B // (_NW * _CHUNK)
_V = 1000000
_RC = 32768
_RQ = _RC // 4
_NBLK = (_V + _RC - 1) // _RC
_QV = _NBLK * _RQ


def _relayout_tc(tableT, eye):
  def body(x_ref, eye_ref, out_ref):
    xb = x_ref[...].astype(jnp.bfloat16)
    xt = lax.dot_general(
        xb, eye_ref[...], (((0,), (0,)), ((), ())),
        preferred_element_type=jnp.float32)
    bi = lax.bitcast_convert_type(xt, jnp.int32)
    out_ref[:, 0:_E] = (
        lax.shift_right_logical(bi[0:_RQ, :], 16) | bi[_RQ:2 * _RQ, :])
    out_ref[:, _E:2 * _E] = (
        lax.shift_right_logical(bi[2 * _RQ:3 * _RQ, :], 16)
        | bi[3 * _RQ:4 * _RQ, :])

  return pl.pallas_call(
      body,
      grid=(_NBLK,),
      in_specs=[
          pl.BlockSpec((_E, _RC), lambda i: (0, i)),
          pl.BlockSpec((_E, _E), lambda i: (0, 0)),
      ],
      out_specs=pl.BlockSpec((_RQ, 2 * _E), lambda i: (i, 0)),
      out_shape=jax.ShapeDtypeStruct((_QV, 2 * _E), jnp.int32),
      compiler_params=pltpu.CompilerParams(
          dimension_semantics=("parallel",)),
  )(tableT, eye)


def _gather_sc(idx2d, paired):
  mesh = plsc.VectorSubcoreMesh(core_axis_name="c", subcore_axis_name="s")

  @functools.partial(
      pl.kernel,
      mesh=mesh,
      out_type=jax.ShapeDtypeStruct((_B // _CHUNK, _CHUNK, 2 * _E),
                                    jnp.int32),
      scratch_types=[
          pltpu.VMEM((_CPW, _CHUNK), jnp.int32),
          pltpu.VMEM((_CPW, _CHUNK, 2 * _E), jnp.int32),
          pltpu.SemaphoreType.DMA,
      ],
  )
  def gk(idx_hbm, table_hbm, out_hbm, idx_v, rows_v, sem):
    wid = lax.axis_index("s") * _NC + lax.axis_index("c")
    base = wid * _CPW
    pltpu.sync_copy(idx_hbm.at[pl.ds(base, _CPW)], idx_v)
    copies = [
        pltpu.async_copy(table_hbm.at[idx_v.at[j]], rows_v.at[j], sem)
        for j in range(_CPW)
    ]
    for c in copies:
      c.wait()
    pltpu.sync_copy(rows_v, out_hbm.at[pl.ds(base, _CPW)])

  return gk(idx2d, paired)


_T = 2048


def _k1_body(feat, w1, b1, h_ref, st_ref):
  i = pl.program_id(0)
  h = jnp.dot(feat[...], w1[...], preferred_element_type=jnp.float32) + b1[...]
  h_ref[...] = h
  st = jnp.concatenate(
      [jnp.sum(h, axis=0, keepdims=True),
       jnp.sum(h * h, axis=0, keepdims=True)], axis=0)

  @pl.when(i == 0)
  def _():
    st_ref[...] = st

  @pl.when(i != 0)
  def _():
    st_ref[...] += st


def _k2_body(emb2, par, h, w2a, w2b, b2, st1, g1, be1, z_ref, st_ref):
  i = pl.program_id(0)
  inv_b = 1.0 / _B
  m1 = st1[0:1, :] * inv_b
  v1 = st1[1:2, :] * inv_b - m1 * m1
  sc1 = g1[...] * lax.rsqrt(v1 + _EPS)
  sh1 = be1[...] - m1 * sc1
  hbn = h[...] * sc1 + sh1
  w = emb2[...]
  q = par[...]
  half = jnp.where(q < 1.5, w[:, 0:_E], w[:, _E:2 * _E])
  vlo = lax.bitcast_convert_type(jnp.left_shift(half, 16), jnp.float32)
  vhi = lax.bitcast_convert_type(half & jnp.int32(-65536), jnp.float32)
  p = q - 2.0 * jnp.floor(q * 0.5)
  emb = jnp.where(p > 0.5, vhi, vlo)
  z = (jnp.dot(emb, w2a[...], preferred_element_type=jnp.float32)
       + jnp.dot(hbn, w2b[...], preferred_element_type=jnp.float32)
       + b2[...])
  z_ref[...] = z
  st = jnp.concatenate(
      [jnp.sum(z, axis=0, keepdims=True),
       jnp.sum(z * z, axis=0, keepdims=True)], axis=0)

  @pl.when(i == 0)
  def _():
    st_ref[...] = st

  @pl.when(i != 0)
  def _():
    st_ref[...] += st


def _k3_body(z2, st2, g2, be2, w3, out_ref):
  inv_b = 1.0 / _B
  m2 = st2[0:1, :] * inv_b
  v2 = st2[1:2, :] * inv_b - m2 * m2
  sc2 = g2[...] * lax.rsqrt(v2 + _EPS)
  sh2 = be2[...] - m2 * sc2
  z = z2[...] * sc2 + sh2
  g = 0.5 * z * (1.0 + lax.erf(z * 0.7071067811865476))
  out_ref[...] = jnp.dot(g, w3[...], preferred_element_type=jnp.float32)


def kernel(entity_ids, features, table, W1, b1, g1, be1, W2, b2, g2, be2, W3):
  idx = entity_ids.astype(jnp.int32)
  paired = _relayout_tc(table.T, jnp.eye(_E, dtype=jnp.bfloat16))
  idxq = ((idx // _RC) * _RQ + (idx % _RQ)).reshape(_B // _CHUNK, _CHUNK)
  emb2 = _gather_sc(idxq, paired).reshape(_B, 2 * _E)
  par = ((idx // _RQ) % 4).astype(jnp.float32).reshape(_B, 1)

  b1r = b1.reshape(1, _H)
  g1r = g1.reshape(1, _H)
  be1r = be1.reshape(1, _H)
  b2r = b2.reshape(1, _H)
  g2r = g2.reshape(1, _H)
  be2r = be2.reshape(1, _H)
  w2a = W2[:_E]
  w2b = W2[_E:]

  grid = (_B // _T,)

  h, st1 = pl.pallas_call(
      _k1_body,
      grid=grid,
      in_specs=[
          pl.BlockSpec((_T, _F), lambda i: (i, 0)),
          pl.BlockSpec((_F, _H), lambda i: (0, 0)),
          pl.BlockSpec((1, _H), lambda i: (0, 0)),
      ],
      out_specs=[
          pl.BlockSpec((_T, _H), lambda i: (i, 0)),
          pl.BlockSpec((2, _H), lambda i: (0, 0)),
      ],
      out_shape=[
          jax.ShapeDtypeStruct((_B, _H), jnp.float32),
          jax.ShapeDtypeStruct((2, _H), jnp.float32),
      ],
  )(features, W1, b1r)

  z2, st2 = pl.pallas_call(
      _k2_body,
      grid=grid,
      in_specs=[
          pl.BlockSpec((_T, 2 * _E), lambda i: (i, 0)),
          pl.BlockSpec((_T, 1), lambda i: (i, 0)),
          pl.BlockSpec((_T, _H), lambda i: (i, 0)),
          pl.BlockSpec((_E, _H), lambda i: (0, 0)),
          pl.BlockSpec((_H, _H), lambda i: (0, 0)),
          pl.BlockSpec((1, _H), lambda i: (0, 0)),
          pl.BlockSpec((2, _H), lambda i: (0, 0)),
          pl.BlockSpec((1, _H), lambda i: (0, 0)),
          pl.BlockSpec((1, _H), lambda i: (0, 0)),
      ],
      out_specs=[
          pl.BlockSpec((_T, _H), lambda i: (i, 0)),
          pl.BlockSpec((2, _H), lambda i: (0, 0)),
      ],
      out_shape=[
          jax.ShapeDtypeStruct((_B, _H), jnp.float32),
          jax.ShapeDtypeStruct((2, _H), jnp.float32),
      ],
  )(emb2, par, h, w2a, w2b, b2r, st1, g1r, be1r)

  out = pl.pallas_call(
      _k3_body,
      grid=grid,
      in_specs=[
          pl.BlockSpec((_T, _H), lambda i: (i, 0)),
          pl.BlockSpec((2, _H), lambda i: (0, 0)),
          pl.BlockSpec((1, _H), lambda i: (0, 0)),
          pl.BlockSpec((1, _H), lambda i: (0, 0)),
          pl.BlockSpec((_H, _O), lambda i: (0, 0)),
      ],
      out_specs=pl.BlockSpec((_T, _O), lambda i: (i, 0)),
      out_shape=jax.ShapeDtypeStruct((_B, _O), jnp.float32),
  )(z2, st2, g2r, be2r, W3)

  return out


# speedup vs baseline: 1.0278x; 1.0278x over previous
import functools

import jax
import jax.numpy as jnp
from jax import lax
from jax.experimental import pallas as pl
from jax.experimental.pallas import tpu as pltpu
from jax.experimental.pallas import tpu_sc as plsc

_EPS = 1e-5

_B = 16384
_E = 64
_F = 128
_H = 256
_O = 128


_NC = 2
_NS = 16
_NW = _NC * _NS
_CHUNK = 128
_CPW = _B // (_NW * _CHUNK)
_V = 1000000
_RC = 32768
_RQ = _RC // 4
_NBLK = (_V + _RC - 1) // _RC
_QV = _NBLK * _RQ


def _relayout_tc(tableT, eye):
  def body(x_ref, eye_ref, out_ref):
    xb = x_ref[...].astype(jnp.bfloat16)
    xt = lax.dot_general(
        xb, eye_ref[...], (((0,), (0,)), ((), ())),
        preferred_element_type=jnp.float32)
    bi = lax.bitcast_convert_type(xt, jnp.int32)
    out_ref[:, 0:_E] = (
        lax.shift_right_logical(bi[0:_RQ, :], 16) | bi[_RQ:2 * _RQ, :])
    out_ref[:, _E:2 * _E] = (
        lax.shift_right_logical(bi[2 * _RQ:3 * _RQ, :], 16)
        | bi[3 * _RQ:4 * _RQ, :])

  return pl.pallas_call(
      body,
      grid=(_NBLK,),
      in_specs=[
          pl.BlockSpec((_E, _RC), lambda i: (0, i)),
          pl.BlockSpec((_E, _E), lambda i: (0, 0)),
      ],
      out_specs=pl.BlockSpec((_RQ, 2 * _E), lambda i: (i, 0)),
      out_shape=jax.ShapeDtypeStruct((_QV, 2 * _E), jnp.int32),
      compiler_params=pltpu.CompilerParams(
          dimension_semantics=("parallel",)),
  )(tableT, eye)


def _gather_sc(idx2d, paired):
  mesh = plsc.VectorSubcoreMesh(core_axis_name="c", subcore_axis_name="s")

  @functools.partial(
      pl.kernel,
      mesh=mesh,
      out_type=jax.ShapeDtypeStruct((_B // _CHUNK, _CHUNK, 2 * _E),
                                    jnp.int32),
      scratch_types=[
          pltpu.VMEM((_CPW, _CHUNK), jnp.int32),
          pltpu.VMEM((_CPW, _CHUNK, 2 * _E), jnp.int32),
          pltpu.SemaphoreType.DMA,
      ],
  )
  def gk(idx_hbm, table_hbm, out_hbm, idx_v, rows_v, sem):
    wid = lax.axis_index("s") * _NC + lax.axis_index("c")
    base = wid * _CPW
    pltpu.sync_copy(idx_hbm.at[pl.ds(base, _CPW)], idx_v)
    copies = [
        pltpu.async_copy(table_hbm.at[idx_v.at[j]], rows_v.at[j], sem)
        for j in range(_CPW)
    ]
    for c in copies:
      c.wait()
    pltpu.sync_copy(rows_v, out_hbm.at[pl.ds(base, _CPW)])

  return gk(idx2d, paired)


_T = 2048


def _k1_body(feat, w1, b1, h_ref, st_ref):
  i = pl.program_id(0)
  h = jnp.dot(feat[...], w1[...], preferred_element_type=jnp.float32) + b1[...]
  h_ref[...] = h.astype(jnp.bfloat16)
  st = jnp.concatenate(
      [jnp.sum(h, axis=0, keepdims=True),
       jnp.sum(h * h, axis=0, keepdims=True)], axis=0)

  @pl.when(i == 0)
  def _():
    st_ref[...] = st

  @pl.when(i != 0)
  def _():
    st_ref[...] += st


def _k2_body(emb2, par, h, w2a, w2b, b2, st1, g1, be1, z_ref, st_ref):
  i = pl.program_id(0)
  inv_b = 1.0 / _B
  m1 = st1[0:1, :] * inv_b
  v1 = st1[1:2, :] * inv_b - m1 * m1
  sc1 = g1[...] * lax.rsqrt(v1 + _EPS)
  sh1 = be1[...] - m1 * sc1
  hbn = h[...].astype(jnp.float32) * sc1 + sh1
  w = emb2[...]
  q = par[...]
  half = jnp.where(q < 1.5, w[:, 0:_E], w[:, _E:2 * _E])
  vlo = lax.bitcast_convert_type(jnp.left_shift(half, 16), jnp.float32)
  vhi = lax.bitcast_convert_type(half & jnp.int32(-65536), jnp.float32)
  p = q - 2.0 * jnp.floor(q * 0.5)
  emb = jnp.where(p > 0.5, vhi, vlo)
  z = (jnp.dot(emb, w2a[...], preferred_element_type=jnp.float32)
       + jnp.dot(hbn, w2b[...], preferred_element_type=jnp.float32)
       + b2[...])
  z_ref[...] = z.astype(jnp.bfloat16)
  st = jnp.concatenate(
      [jnp.sum(z, axis=0, keepdims=True),
       jnp.sum(z * z, axis=0, keepdims=True)], axis=0)

  @pl.when(i == 0)
  def _():
    st_ref[...] = st

  @pl.when(i != 0)
  def _():
    st_ref[...] += st


def _k3_body(z2, st2, g2, be2, w3, out_ref):
  inv_b = 1.0 / _B
  m2 = st2[0:1, :] * inv_b
  v2 = st2[1:2, :] * inv_b - m2 * m2
  sc2 = g2[...] * lax.rsqrt(v2 + _EPS)
  sh2 = be2[...] - m2 * sc2
  z = z2[...].astype(jnp.float32) * sc2 + sh2
  g = 0.5 * z * (1.0 + lax.erf(z * 0.7071067811865476))
  out_ref[...] = jnp.dot(g, w3[...], preferred_element_type=jnp.float32)


def kernel(entity_ids, features, table, W1, b1, g1, be1, W2, b2, g2, be2, W3):
  idx = entity_ids.astype(jnp.int32)
  paired = _relayout_tc(table.T, jnp.eye(_E, dtype=jnp.bfloat16))
  idxq = ((idx // _RC) * _RQ + (idx % _RQ)).reshape(_B // _CHUNK, _CHUNK)
  emb2 = _gather_sc(idxq, paired).reshape(_B, 2 * _E)
  par = ((idx // _RQ) % 4).astype(jnp.float32).reshape(_B, 1)

  b1r = b1.reshape(1, _H)
  g1r = g1.reshape(1, _H)
  be1r = be1.reshape(1, _H)
  b2r = b2.reshape(1, _H)
  g2r = g2.reshape(1, _H)
  be2r = be2.reshape(1, _H)
  w2a = W2[:_E]
  w2b = W2[_E:]

  grid = (_B // _T,)

  h, st1 = pl.pallas_call(
      _k1_body,
      grid=grid,
      in_specs=[
          pl.BlockSpec((_T, _F), lambda i: (i, 0)),
          pl.BlockSpec((_F, _H), lambda i: (0, 0)),
          pl.BlockSpec((1, _H), lambda i: (0, 0)),
      ],
      out_specs=[
          pl.BlockSpec((_T, _H), lambda i: (i, 0)),
          pl.BlockSpec((2, _H), lambda i: (0, 0)),
      ],
      out_shape=[
          jax.ShapeDtypeStruct((_B, _H), jnp.bfloat16),
          jax.ShapeDtypeStruct((2, _H), jnp.float32),
      ],
  )(features, W1, b1r)

  z2, st2 = pl.pallas_call(
      _k2_body,
      grid=grid,
      in_specs=[
          pl.BlockSpec((_T, 2 * _E), lambda i: (i, 0)),
          pl.BlockSpec((_T, 1), lambda i: (i, 0)),
          pl.BlockSpec((_T, _H), lambda i: (i, 0)),
          pl.BlockSpec((_E, _H), lambda i: (0, 0)),
          pl.BlockSpec((_H, _H), lambda i: (0, 0)),
          pl.BlockSpec((1, _H), lambda i: (0, 0)),
          pl.BlockSpec((2, _H), lambda i: (0, 0)),
          pl.BlockSpec((1, _H), lambda i: (0, 0)),
          pl.BlockSpec((1, _H), lambda i: (0, 0)),
      ],
      out_specs=[
          pl.BlockSpec((_T, _H), lambda i: (i, 0)),
          pl.BlockSpec((2, _H), lambda i: (0, 0)),
      ],
      out_shape=[
          jax.ShapeDtypeStruct((_B, _H), jnp.bfloat16),
          jax.ShapeDtypeStruct((2, _H), jnp.float32),
      ],
  )(emb2, par, h, w2a, w2b, b2r, st1, g1r, be1r)

  out = pl.pallas_call(
      _k3_body,
      grid=grid,
      in_specs=[
          pl.BlockSpec((_T, _H), lambda i: (i, 0)),
          pl.BlockSpec((2, _H), lambda i: (0, 0)),
          pl.BlockSpec((1, _H), lambda i: (0, 0)),
          pl.BlockSpec((1, _H), lambda i: (0, 0)),
          pl.BlockSpec((_H, _O), lambda i: (0, 0)),
      ],
      out_specs=pl.BlockSpec((_T, _O), lambda i: (i, 0)),
      out_shape=jax.ShapeDtypeStruct((_B, _O), jnp.float32),
  )(z2, st2, g2r, be2r, W3)

  return out
